# speedup vs baseline: 1.2430x; 1.2430x over previous
import jax
import jax.numpy as jnp
from jax.experimental import pallas as pl
from jax.experimental.pallas import tpu as pltpu


def _prop_block(adj_ref, v_ref, o_ref):
    o_ref[...] = jnp.dot(adj_ref[...], v_ref[...],
                         preferred_element_type=jnp.float32)


def _pick_bm(n):
    for bm in (400, 200, 100, 50, 8):
        if n % bm == 0:
            return bm
    return n


def _propagate(adj, v):
    n = adj.shape[0]
    d = v.shape[1]
    bm = _pick_bm(n)
    return pl.pallas_call(
        _prop_block,
        grid=(n // bm,),
        in_specs=[
            pl.BlockSpec((bm, n), lambda i: (i, 0)),
            pl.BlockSpec((n, d), lambda i: (0, 0)),
        ],
        out_specs=pl.BlockSpec((bm, d), lambda i: (i, 0)),
        out_shape=jax.ShapeDtypeStruct((n, d), jnp.float32),
        compiler_params=pltpu.CompilerParams(
            dimension_semantics=("arbitrary",),
        ),
    )(adj, v)


def _linear_block(h_ref, wt_ref, b_ref, o_ref):
    o_ref[...] = jnp.dot(h_ref[...], wt_ref[...],
                         preferred_element_type=jnp.float32) + b_ref[...]


def _linear(h, wt, b2):
    n, d_in = h.shape
    d_out = wt.shape[1]
    bm = _pick_bm(n)
    return pl.pallas_call(
        _linear_block,
        grid=(n // bm,),
        in_specs=[
            pl.BlockSpec((bm, d_in), lambda i: (i, 0)),
            pl.BlockSpec((d_in, d_out), lambda i: (0, 0)),
            pl.BlockSpec((1, d_out), lambda i: (0, 0)),
        ],
        out_specs=pl.BlockSpec((bm, d_out), lambda i: (i, 0)),
        out_shape=jax.ShapeDtypeStruct((n, d_out), jnp.float32),
        compiler_params=pltpu.CompilerParams(
            dimension_semantics=("arbitrary",),
        ),
    )(h, wt, b2)


def kernel(x, adj, W, b, k):
    h = jax.lax.fori_loop(0, k, lambda i, h: _propagate(adj, h), x)
    return _linear(h, W.T, b.reshape(1, -1))


# speedup vs baseline: 1.3222x; 1.0637x over previous
import jax
import jax.numpy as jnp
from jax.experimental import pallas as pl
from jax.experimental.pallas import tpu as pltpu


def _prop_block(adj_ref, v_ref, o_ref):
    o_ref[...] = jnp.dot(adj_ref[...], v_ref[...],
                         preferred_element_type=jnp.float32)


def _pick_bm(n):
    for bm in (400, 200, 100, 50, 8):
        if n % bm == 0:
            return bm
    return n


def _propagate(adj, v):
    n = adj.shape[0]
    d = v.shape[1]
    bm = _pick_bm(n)
    return pl.pallas_call(
        _prop_block,
        grid=(n // bm,),
        in_specs=[
            pl.BlockSpec((bm, n), lambda i: (i, 0)),
            pl.BlockSpec((n, d), lambda i: (0, 0)),
        ],
        out_specs=pl.BlockSpec((bm, d), lambda i: (i, 0)),
        out_shape=jax.ShapeDtypeStruct((n, d), jnp.float32),
        compiler_params=pltpu.CompilerParams(
            dimension_semantics=("arbitrary",),
        ),
    )(adj, v)


def _prop_linear_block(adj_ref, v_ref, wt_ref, b_ref, o_ref):
    h = jnp.dot(adj_ref[...], v_ref[...], preferred_element_type=jnp.float32)
    o_ref[...] = jnp.dot(h, wt_ref[...],
                         preferred_element_type=jnp.float32) + b_ref[...]


def _propagate_linear(adj, v, wt, b2):
    n = adj.shape[0]
    d = v.shape[1]
    d_out = wt.shape[1]
    bm = _pick_bm(n)
    return pl.pallas_call(
        _prop_linear_block,
        grid=(n // bm,),
        in_specs=[
            pl.BlockSpec((bm, n), lambda i: (i, 0)),
            pl.BlockSpec((n, d), lambda i: (0, 0)),
            pl.BlockSpec((d, d_out), lambda i: (0, 0)),
            pl.BlockSpec((1, d_out), lambda i: (0, 0)),
        ],
        out_specs=pl.BlockSpec((bm, d_out), lambda i: (i, 0)),
        out_shape=jax.ShapeDtypeStruct((n, d_out), jnp.float32),
        compiler_params=pltpu.CompilerParams(
            dimension_semantics=("arbitrary",),
        ),
    )(adj, v, wt, b2)


def kernel(x, adj, W, b, k):
    h = jax.lax.fori_loop(0, k - 1, lambda i, h: _propagate(adj, h), x)
    return _propagate_linear(adj, h, W.T, b.reshape(1, -1))
